# baseline (speedup 1.0000x reference)
import functools

import jax
import jax.numpy as jnp
import numpy as np
from jax import lax
from jax.experimental import pallas as pl
from jax.experimental.pallas import tpu as pltpu
from jax.experimental.pallas import tpu_sc as plsc

B = 4096
C = 1000

NC = 2
NS = 16
NW = NC * NS
RPT = B // NW
CH = 16
NCH = RPT // CH
K = 20.0
UNROLL = 8


def _gen_offset():
    return jax.random.randint(jax.random.key(42), (B,), 1, C, dtype=jnp.int32)


try:
    try:
        with jax.default_device(jax.local_devices(backend="cpu")[0]):
            _OFFSET = np.asarray(_gen_offset())
    except Exception:
        _OFFSET = np.asarray(_gen_offset())
except Exception:
    _OFFSET = None


def _fixed_offset():
    return _gen_offset() if _OFFSET is None else jnp.asarray(_OFFSET)


_sc_mesh = plsc.VectorSubcoreMesh(core_axis_name="c", subcore_axis_name="s")


@functools.partial(
    pl.kernel,
    mesh=_sc_mesh,
    out_type=jax.ShapeDtypeStruct((B,), jnp.float32),
    scratch_types=[
        pltpu.VMEM((CH, C), jnp.float32),
        pltpu.VMEM((CH, C), jnp.float32),
        pltpu.VMEM((RPT,), jnp.int32),
        pltpu.VMEM((RPT,), jnp.int32),
        pltpu.VMEM((RPT,), jnp.float32),
        pltpu.SemaphoreType.DMA,
        pltpu.SemaphoreType.DMA,
    ],
    compiler_params=pltpu.CompilerParams(
        use_tc_tiling_on_sc=True, needs_layout_passes=False
    ),
)
def _sc_loss(x_hbm, tgt_hbm, off_hbm, ct_hbm,
             buf0, buf1, tgt_v, off_v, ct_v, sem0, sem1):
    wid = lax.axis_index("s") * NC + lax.axis_index("c")
    base = wid * RPT
    pltpu.sync_copy(tgt_hbm.at[pl.ds(base, RPT)], tgt_v)
    pltpu.sync_copy(off_hbm.at[pl.ds(base, RPT)], off_v)
    bufs = (buf0, buf1)
    sems = (sem0, sem1)
    r16 = lax.iota(jnp.int32, 16)

    cps = [pltpu.async_copy(x_hbm.at[pl.ds(base, CH)], buf0, sem0)]
    for c in range(NCH):
        buf = bufs[c % 2]
        if c + 1 < NCH:
            cps.append(pltpu.async_copy(
                x_hbm.at[pl.ds(base + (c + 1) * CH, CH)],
                bufs[(c + 1) % 2], sems[(c + 1) % 2]))
        cps[c].wait()

        def colstep(jj, z):
            j0 = jj * UNROLL
            for u in range(UNROLL):
                jv = jnp.broadcast_to(j0 + u, (16,)).astype(jnp.int32)
                col = plsc.load_gather(buf, [r16, jv])
                z = z + jnp.exp(col - K)
            return z

        z = lax.fori_loop(0, C // UNROLL, colstep, jnp.zeros((16,), jnp.float32))

        t = tgt_v[pl.ds(c * CH, 16)]
        o = off_v[pl.ds(c * CH, 16)]
        n = lax.rem(t + o, C)
        oy = plsc.load_gather(buf, [r16, t])
        ok = plsc.load_gather(buf, [r16, n])
        wy = jnp.minimum(jnp.maximum(jnp.exp(oy - K) / z, 1e-7), 1.0)
        wk = jnp.minimum(jnp.maximum(jnp.exp(ok - K) / z, 1e-7), 1.0)
        tt = 1.0 - (wk - wy)
        gneg = -(wk * (wy + wk)) * tt - wk * (1.0 - wk) * tt
        gpos = wk * tt + wk * wy * tt
        ct_v[pl.ds(c * CH, 16)] = gneg * ok + gpos * oy

    pltpu.sync_copy(ct_v, ct_hbm.at[pl.ds(base, RPT)])


def _red_body(ct_ref, out_ref):
    out_ref[...] = (-jnp.sum(ct_ref[...]) / B).reshape(1, 1)


def kernel(output, target):
    ct = _sc_loss(output, target, _fixed_offset())
    out = pl.pallas_call(
        _red_body,
        in_specs=[pl.BlockSpec((32, 128), lambda: (0, 0))],
        out_specs=pl.BlockSpec((1, 1), lambda: (0, 0)),
        out_shape=jax.ShapeDtypeStruct((1, 1), jnp.float32),
    )(ct.reshape(32, 128))
    return out[0, 0]


# speedup vs baseline: 2.0383x; 2.0383x over previous
import functools

import jax
import jax.numpy as jnp
import numpy as np
from jax import lax
from jax.experimental import pallas as pl
from jax.experimental.pallas import tpu as pltpu
from jax.experimental.pallas import tpu_sc as plsc

B = 4096
C = 1000

NC = 2
NS = 16
NW = NC * NS
RPT = B // NW
CH = 16
NCH = RPT // CH
K = 20.0
UNROLL = 8


def _gen_offset():
    return jax.random.randint(jax.random.key(42), (B,), 1, C, dtype=jnp.int32)


try:
    try:
        with jax.default_device(jax.local_devices(backend="cpu")[0]):
            _OFFSET = np.asarray(_gen_offset())
    except Exception:
        _OFFSET = np.asarray(_gen_offset())
except Exception:
    _OFFSET = None


def _fixed_offset():
    return _gen_offset() if _OFFSET is None else jnp.asarray(_OFFSET)


_sc_mesh = plsc.VectorSubcoreMesh(core_axis_name="c", subcore_axis_name="s")


@functools.partial(
    pl.kernel,
    mesh=_sc_mesh,
    out_type=jax.ShapeDtypeStruct((B,), jnp.float32),
    scratch_types=[
        pltpu.VMEM((CH, C), jnp.float32),
        pltpu.VMEM((CH, C), jnp.float32),
        pltpu.VMEM((RPT,), jnp.int32),
        pltpu.VMEM((RPT,), jnp.int32),
        pltpu.VMEM((RPT,), jnp.float32),
        pltpu.SemaphoreType.DMA,
        pltpu.SemaphoreType.DMA,
    ],
    compiler_params=pltpu.CompilerParams(
        use_tc_tiling_on_sc=True, needs_layout_passes=False
    ),
)
def _sc_loss(x_hbm, tgt_hbm, off_hbm, ct_hbm,
             buf0, buf1, tgt_v, off_v, ct_v, sem0, sem1):
    wid = lax.axis_index("s") * NC + lax.axis_index("c")
    base = wid * RPT
    pltpu.sync_copy(tgt_hbm.at[pl.ds(base, RPT)], tgt_v)
    pltpu.sync_copy(off_hbm.at[pl.ds(base, RPT)], off_v)
    bufs = (buf0, buf1)
    sems = (sem0, sem1)
    r16 = lax.iota(jnp.int32, 16)

    cps = [pltpu.async_copy(x_hbm.at[pl.ds(base, CH)], buf0, sem0)]
    for c in range(NCH):
        buf = bufs[c % 2]
        if c + 1 < NCH:
            cps.append(pltpu.async_copy(
                x_hbm.at[pl.ds(base + (c + 1) * CH, CH)],
                bufs[(c + 1) % 2], sems[(c + 1) % 2]))
        cps[c].wait()

        def rowstep(r, z, buf=buf):
            accs = [jnp.zeros((16,), jnp.float32) for _ in range(4)]
            for j in range(62):
                accs[j % 4] = accs[j % 4] + jnp.exp(buf[r, pl.ds(j * 16, 16)] - K)
            tail = jnp.exp(buf[r, pl.ds(C - 16, 16)] - K)
            accs[2] = accs[2] + jnp.where(r16 >= (992 - (C - 16)), tail, 0.0)
            total = jnp.sum((accs[0] + accs[1]) + (accs[2] + accs[3]))
            return jnp.where(r16 == r, total, z)

        z = lax.fori_loop(0, CH, rowstep, jnp.zeros((16,), jnp.float32))

        t = tgt_v[pl.ds(c * CH, 16)]
        o = off_v[pl.ds(c * CH, 16)]
        n = lax.rem(t + o, C)
        oy = plsc.load_gather(buf, [r16, t])
        ok = plsc.load_gather(buf, [r16, n])
        wy = jnp.minimum(jnp.maximum(jnp.exp(oy - K) / z, 1e-7), 1.0)
        wk = jnp.minimum(jnp.maximum(jnp.exp(ok - K) / z, 1e-7), 1.0)
        tt = 1.0 - (wk - wy)
        gneg = -(wk * (wy + wk)) * tt - wk * (1.0 - wk) * tt
        gpos = wk * tt + wk * wy * tt
        ct_v[pl.ds(c * CH, 16)] = gneg * ok + gpos * oy

    pltpu.sync_copy(ct_v, ct_hbm.at[pl.ds(base, RPT)])


def _red_body(ct_ref, out_ref):
    out_ref[...] = (-jnp.sum(ct_ref[...]) / B).reshape(1, 1)


def kernel(output, target):
    ct = _sc_loss(output, target, _fixed_offset())
    out = pl.pallas_call(
        _red_body,
        in_specs=[pl.BlockSpec((32, 128), lambda: (0, 0))],
        out_specs=pl.BlockSpec((1, 1), lambda: (0, 0)),
        out_shape=jax.ShapeDtypeStruct((1, 1), jnp.float32),
    )(ct.reshape(32, 128))
    return out[0, 0]


# speedup vs baseline: 3.1017x; 1.5217x over previous
import jax
import jax.numpy as jnp
import numpy as np
from jax import lax
from jax.experimental import pallas as pl

B = 4096
C = 1000
BLK = 1024
GRID = B // BLK


def _gen_offset():
    return jax.random.randint(jax.random.key(42), (B,), 1, C, dtype=jnp.int32)


try:
    try:
        with jax.default_device(jax.local_devices(backend="cpu")[0]):
            _OFFSET = np.asarray(_gen_offset())
    except Exception:
        _OFFSET = np.asarray(_gen_offset())
except Exception:
    _OFFSET = None


def _fixed_offset():
    return _gen_offset() if _OFFSET is None else jnp.asarray(_OFFSET)


_NWIN = 7


def _gather128(x, idx):
    win = jnp.minimum(idx >> 7, _NWIN)
    acc = jnp.zeros((BLK, 128), jnp.float32)
    for k in range(_NWIN + 1):
        start = 128 * k if k < _NWIN else C - 128
        acc = jnp.where(win == k, x[:, start:start + 128], acc)
    start_of = jnp.where(win == _NWIN, C - 128, win << 7)
    lane = idx - start_of
    cols = lax.broadcasted_iota(jnp.int32, (BLK, 128), 1)
    return jnp.sum(jnp.where(cols == lane, acc, 0.0), axis=1, keepdims=True)


def _body(x_ref, t_ref, o_ref, out_ref):
    i = pl.program_id(0)
    x = x_ref[...]
    t = t_ref[0, 0, :].reshape(BLK, 1)
    n = lax.rem(t + o_ref[0, 0, :].reshape(BLK, 1), C)

    m = jnp.max(x, axis=1, keepdims=True)
    z = jnp.sum(jnp.exp(x - m), axis=1, keepdims=True)
    lz = m + jnp.log(z)

    oy = _gather128(x, t)
    ok = _gather128(x, n)
    wy = jnp.clip(jnp.exp(oy - lz), 1e-7, 1.0)
    wk = jnp.clip(jnp.exp(ok - lz), 1e-7, 1.0)

    tt = 1.0 - (wk - wy)
    gneg = -(wk * (wy + wk)) * tt - wk * (1.0 - wk) * tt
    gpos = wk * tt + wk * wy * tt
    partial = jnp.sum(gneg * ok + gpos * oy).reshape(1, 1)

    prev = jnp.where(i == 0, jnp.zeros((1, 1), jnp.float32), out_ref[...])
    tot = prev + partial
    out_ref[...] = jnp.where(i == GRID - 1, -tot / B, tot)


def kernel(output, target):
    offset3 = _fixed_offset().reshape(GRID, 1, BLK)
    out = pl.pallas_call(
        _body,
        grid=(GRID,),
        in_specs=[
            pl.BlockSpec((BLK, C), lambda i: (i, 0)),
            pl.BlockSpec((1, 1, BLK), lambda i: (i, 0, 0)),
            pl.BlockSpec((1, 1, BLK), lambda i: (i, 0, 0)),
        ],
        out_specs=pl.BlockSpec((1, 1), lambda i: (0, 0)),
        out_shape=jax.ShapeDtypeStruct((1, 1), jnp.float32),
    )(output, target.reshape(GRID, 1, BLK), offset3)
    return out[0, 0]
